# baseline (speedup 1.0000x reference)
import functools
import math

import jax
import jax.numpy as jnp
import numpy as np
from jax.experimental import pallas as pl
from jax.experimental.pallas import tpu as pltpu

K = 16
O = 8
N = 384
NF = 193
NFP = 256


def _dft_constants():
    n = np.arange(N, dtype=np.int64)
    k = np.arange(NFP, dtype=np.int64)
    ang = -2.0 * np.pi * ((n[:, None] * k[None, :]) % N).astype(np.float64) / N
    scale = 1.0 / N
    wr = np.cos(ang) * scale
    wi = np.sin(ang) * scale
    wr[:, NF:] = 0.0
    wi[:, NF:] = 0.0
    m = np.arange(N, dtype=np.int64)
    ang2 = -2.0 * np.pi * ((n[:, None] * m[None, :]) % N).astype(np.float64) / N
    fr = np.cos(ang2)
    fi = np.sin(ang2)
    return (wr.astype(np.float32), wi.astype(np.float32),
            fr.astype(np.float32), fi.astype(np.float32),
            (fr + fi).astype(np.float32))


_BF16 = jnp.bfloat16


_WR, _WI, _FR, _FI, _FRPI = _dft_constants()


def _spec_kernel(*refs, cpb):
    x_refs = refs[:cpb]
    wrwi_ref, fr_ref, fi_ref, frpi_ref, out_ref = refs[cpb:]
    c = pl.program_id(1)
    ar_parts, ai_parts, s_parts = [], [], []
    for j in range(cpb):
        xi = x_refs[j][0, 0].astype(_BF16)
        aa = jnp.dot(xi, wrwi_ref[...], preferred_element_type=jnp.float32)
        s = aa[:, :NFP] + aa[:, NFP:]
        aab = aa.astype(_BF16)
        ar_parts.append(aab[:, :NFP])
        ai_parts.append(aab[:, NFP:])
        s_parts.append(s.astype(_BF16))
    ar_cat = jnp.concatenate(ar_parts, axis=1)
    ai_cat = jnp.concatenate(ai_parts, axis=1)
    s_cat = jnp.concatenate(s_parts, axis=1)
    t1 = jnp.dot(fr_ref[...], ar_cat, preferred_element_type=jnp.float32)
    t2 = jnp.dot(fi_ref[...], ai_cat, preferred_element_type=jnp.float32)
    t3 = jnp.dot(frpi_ref[...], s_cat, preferred_element_type=jnp.float32)
    yr = t1 - t2
    yi = t3 - t1 - t2
    mag = jnp.sqrt(yr * yr + yi * yi)
    msum = mag[:, :NFP]
    for j in range(1, cpb):
        msum = msum + mag[:, j * NFP:(j + 1) * NFP]

    @pl.when(c == 0)
    def _():
        out_ref[0] = msum

    @pl.when(c != 0)
    def _():
        out_ref[0] = out_ref[0] + msum


def _hist_kernel(spec_ref, rb_ref, ob_ref, wt_ref, bvec_ref, out_ref, *, inv_c):
    mag = jnp.log1p(spec_ref[...] * inv_c)
    rb = rb_ref[...]
    ob = ob_ref[...]
    cols = []
    for s in range(K):
        cols.append(jnp.sum(jnp.where(rb == s, mag, 0.0), axis=1, keepdims=True))
    for s in range(O):
        cols.append(jnp.sum(jnp.where(ob == s, mag, 0.0), axis=1, keepdims=True))
    h = jnp.concatenate(cols, axis=1)
    rs = jnp.sum(h[:, :K], axis=1, keepdims=True) + 1e-6
    osum = jnp.sum(h[:, K:], axis=1, keepdims=True) + 1e-6
    col = jax.lax.broadcasted_iota(jnp.int32, (h.shape[0], K + O), 1)
    hn = h / jnp.where(col < K, rs, osum)
    out_ref[...] = (jnp.dot(hn, wt_ref[...], preferred_element_type=jnp.float32)
                    + bvec_ref[...])


def _bin_ids():
    yy, xx = jnp.meshgrid(jnp.linspace(-1.0, 1.0, N),
                          jnp.linspace(0.0, 1.0, NF), indexing='ij')
    rr = jnp.clip(jnp.sqrt(yy ** 2 + xx ** 2), 0.0, 1.0 - 1e-08)
    th = jnp.arctan2(yy, xx + 1e-09) + math.pi / 2
    rb = jnp.clip((rr * K).astype(jnp.int32), 0, K - 1)
    ob = jnp.clip((th / math.pi * O).astype(jnp.int32), 0, O - 1)
    rb = jnp.pad(rb, ((0, 0), (0, NFP - NF)), constant_values=-1)
    ob = jnp.pad(ob, ((0, 0), (0, NFP - NF)), constant_values=-1)
    return rb.reshape(1, N * NFP), ob.reshape(1, N * NFP)


def kernel(x, W, b):
    B, C = x.shape[0], x.shape[1]

    cpb = 4 if C % 4 == 0 else 1
    wrwi = jnp.concatenate([jnp.asarray(_WR, dtype=_BF16),
                            jnp.asarray(_WI, dtype=_BF16)], axis=1)
    spec = pl.pallas_call(
        functools.partial(_spec_kernel, cpb=cpb),
        grid=(B, C // cpb),
        in_specs=[
            pl.BlockSpec((1, 1, N, N),
                         functools.partial(
                             lambda bb, cc, j=0: (bb, cpb * cc + j, 0, 0), j=j))
            for j in range(cpb)
        ] + [
            pl.BlockSpec((N, 2 * NFP), lambda bb, cc: (0, 0)),
            pl.BlockSpec((N, N), lambda bb, cc: (0, 0)),
            pl.BlockSpec((N, N), lambda bb, cc: (0, 0)),
            pl.BlockSpec((N, N), lambda bb, cc: (0, 0)),
        ],
        out_specs=pl.BlockSpec((1, N, NFP), lambda bb, cc: (bb, 0, 0)),
        out_shape=jax.ShapeDtypeStruct((B, N, NFP), jnp.float32),
        compiler_params=pltpu.CompilerParams(
            dimension_semantics=("parallel", "arbitrary")),
    )(*([x] * cpb), wrwi, jnp.asarray(_FR, dtype=_BF16),
      jnp.asarray(_FI, dtype=_BF16), jnp.asarray(_FRPI, dtype=_BF16))

    rb, ob = _bin_ids()
    out = pl.pallas_call(
        functools.partial(_hist_kernel, inv_c=1.0 / C),
        out_shape=jax.ShapeDtypeStruct((B, W.shape[0]), jnp.float32),
    )(spec.reshape(B, N * NFP), rb, ob, W.T, b.reshape(1, -1))
    return out


# speedup vs baseline: 1.0908x; 1.0908x over previous
import functools
import math

import jax
import jax.numpy as jnp
import numpy as np
from jax.experimental import pallas as pl
from jax.experimental.pallas import tpu as pltpu

K = 16
O = 8
N = 384
NF = 193
NFP = 256


def _dft_constants():
    n = np.arange(N, dtype=np.int64)
    k = np.arange(NFP, dtype=np.int64)
    ang = -2.0 * np.pi * ((n[:, None] * k[None, :]) % N).astype(np.float64) / N
    scale = 1.0 / N
    wr = np.cos(ang) * scale
    wi = np.sin(ang) * scale
    wr[:, NF:] = 0.0
    wi[:, NF:] = 0.0
    m = np.arange(N, dtype=np.int64)
    ang2 = -2.0 * np.pi * ((n[:, None] * m[None, :]) % N).astype(np.float64) / N
    fr = np.cos(ang2)
    fi = np.sin(ang2)
    return (wr.astype(np.float32), wi.astype(np.float32),
            fr.astype(np.float32), fi.astype(np.float32),
            (fr + fi).astype(np.float32))


_BF16 = jnp.bfloat16


_WR, _WI, _FR, _FI, _FRPI = _dft_constants()


def _spec_kernel(x_ref, wrwi_ref, fr_ref, fi_ref, frpi_ref, out_ref, *, cpb):
    c = pl.program_id(1)
    ar_parts, ai_parts, s_parts = [], [], []
    for j in range(cpb):
        xi = x_ref[0, j].astype(_BF16)
        aa = jnp.dot(xi, wrwi_ref[...], preferred_element_type=jnp.float32)
        s = aa[:, :NFP] + aa[:, NFP:]
        aab = aa.astype(_BF16)
        ar_parts.append(aab[:, :NFP])
        ai_parts.append(aab[:, NFP:])
        s_parts.append(s.astype(_BF16))
    ar_cat = jnp.concatenate(ar_parts, axis=1)
    ai_cat = jnp.concatenate(ai_parts, axis=1)
    s_cat = jnp.concatenate(s_parts, axis=1)
    t1 = jnp.dot(fr_ref[...], ar_cat, preferred_element_type=jnp.float32)
    t2 = jnp.dot(fi_ref[...], ai_cat, preferred_element_type=jnp.float32)
    t3 = jnp.dot(frpi_ref[...], s_cat, preferred_element_type=jnp.float32)
    yr = t1 - t2
    yi = t3 - t1 - t2
    mag = jnp.sqrt(yr * yr + yi * yi)
    msum = mag[:, :NFP]
    for j in range(1, cpb):
        msum = msum + mag[:, j * NFP:(j + 1) * NFP]

    @pl.when(c == 0)
    def _():
        out_ref[0] = msum

    @pl.when(c != 0)
    def _():
        out_ref[0] = out_ref[0] + msum


def _hist_kernel(spec_ref, rb_ref, ob_ref, wt_ref, bvec_ref, out_ref, *, inv_c):
    mag = jnp.log1p(spec_ref[...] * inv_c)
    rb = rb_ref[...]
    ob = ob_ref[...]
    cols = []
    for s in range(K):
        cols.append(jnp.sum(jnp.where(rb == s, mag, 0.0), axis=1, keepdims=True))
    for s in range(O):
        cols.append(jnp.sum(jnp.where(ob == s, mag, 0.0), axis=1, keepdims=True))
    h = jnp.concatenate(cols, axis=1)
    rs = jnp.sum(h[:, :K], axis=1, keepdims=True) + 1e-6
    osum = jnp.sum(h[:, K:], axis=1, keepdims=True) + 1e-6
    col = jax.lax.broadcasted_iota(jnp.int32, (h.shape[0], K + O), 1)
    hn = h / jnp.where(col < K, rs, osum)
    out_ref[...] = (jnp.dot(hn, wt_ref[...], preferred_element_type=jnp.float32)
                    + bvec_ref[...])


def _bin_ids():
    yy, xx = jnp.meshgrid(jnp.linspace(-1.0, 1.0, N),
                          jnp.linspace(0.0, 1.0, NF), indexing='ij')
    rr = jnp.clip(jnp.sqrt(yy ** 2 + xx ** 2), 0.0, 1.0 - 1e-08)
    th = jnp.arctan2(yy, xx + 1e-09) + math.pi / 2
    rb = jnp.clip((rr * K).astype(jnp.int32), 0, K - 1)
    ob = jnp.clip((th / math.pi * O).astype(jnp.int32), 0, O - 1)
    rb = jnp.pad(rb, ((0, 0), (0, NFP - NF)), constant_values=-1)
    ob = jnp.pad(ob, ((0, 0), (0, NFP - NF)), constant_values=-1)
    return rb.reshape(1, N * NFP), ob.reshape(1, N * NFP)


def kernel(x, W, b):
    B, C = x.shape[0], x.shape[1]

    cpb = 8 if C % 8 == 0 else 1
    wrwi = jnp.concatenate([jnp.asarray(_WR, dtype=_BF16),
                            jnp.asarray(_WI, dtype=_BF16)], axis=1)
    spec = pl.pallas_call(
        functools.partial(_spec_kernel, cpb=cpb),
        grid=(B, C // cpb),
        in_specs=[
            pl.BlockSpec((1, cpb, N, N), lambda bb, cc: (bb, cc, 0, 0)),
            pl.BlockSpec((N, 2 * NFP), lambda bb, cc: (0, 0)),
            pl.BlockSpec((N, N), lambda bb, cc: (0, 0)),
            pl.BlockSpec((N, N), lambda bb, cc: (0, 0)),
            pl.BlockSpec((N, N), lambda bb, cc: (0, 0)),
        ],
        out_specs=pl.BlockSpec((1, N, NFP), lambda bb, cc: (bb, 0, 0)),
        out_shape=jax.ShapeDtypeStruct((B, N, NFP), jnp.float32),
        compiler_params=pltpu.CompilerParams(
            dimension_semantics=("parallel", "arbitrary")),
    )(x, wrwi, jnp.asarray(_FR, dtype=_BF16),
      jnp.asarray(_FI, dtype=_BF16), jnp.asarray(_FRPI, dtype=_BF16))

    rb, ob = _bin_ids()
    out = pl.pallas_call(
        functools.partial(_hist_kernel, inv_c=1.0 / C),
        out_shape=jax.ShapeDtypeStruct((B, W.shape[0]), jnp.float32),
    )(spec.reshape(B, N * NFP), rb, ob, W.T, b.reshape(1, -1))
    return out
